# Initial kernel scaffold; baseline (speedup 1.0000x reference)
import jax
import jax.numpy as jnp
from jax.experimental import pallas as pl


def kernel(script_ids, family_embed, retroflex_bias, W1, b1, W2, b2, Ws, bs, Wsh, bsh):
    raise NotImplementedError("write your pallas kernel here")


# speedup vs baseline: 6.5271x; 6.5271x over previous
import functools

import jax
import jax.numpy as jnp
from jax import lax
from jax.experimental import pallas as pl
from jax.experimental.pallas import tpu as pltpu

N_FAM = 12
SED = 32
ENC = 128
B, L = 16384, 20
LP = 24
BB = 512
GRID = B // BB
M = BB * LP


def _body(ids_ref, fe_ref, rb_ref, w1_ref, b1_ref, w2_ref, b2_ref,
          ws_ref, bs_ref, wsh_ref, bsh_ref,
          proj_ref, scale_ref, shift_ref):
    raw = fe_ref[...] + rb_ref[...]
    h = lax.dot_general(raw, w1_ref[...], (((1,), (1,)), ((), ())),
                        preferred_element_type=jnp.float32) + b1_ref[...]
    h = h * jax.nn.sigmoid(h)
    p = lax.dot_general(h, w2_ref[...], (((1,), (1,)), ((), ())),
                        preferred_element_type=jnp.float32) + b2_ref[...]
    s = lax.dot_general(p, ws_ref[...], (((1,), (1,)), ((), ())),
                        preferred_element_type=jnp.float32) + bs_ref[...]
    sh = lax.dot_general(p, wsh_ref[...], (((1,), (1,)), ((), ())),
                         preferred_element_type=jnp.float32) + bsh_ref[...]
    table = jnp.concatenate([p, s, sh], axis=1)

    ids = ids_ref[...]
    iota = lax.broadcasted_iota(jnp.int32, (N_FAM, M), 0)
    oh = jnp.where(ids == iota, 1.0, 0.0).astype(jnp.float32)
    res = lax.dot_general(oh, table, (((0,), (0,)), ((), ())),
                          preferred_element_type=jnp.float32)
    res3 = res.reshape(BB, LP, 3 * ENC)
    proj_ref[...] = res3[:, :L, 0:ENC]
    scale_ref[...] = res3[:, :L, ENC:2 * ENC]
    shift_ref[...] = res3[:, :L, 2 * ENC:3 * ENC]


@jax.jit
def _run(ids_pad, fe, rb, w1, b1, w2, b2, ws, bs, wsh, bsh):
    full = lambda shape: pl.BlockSpec(shape, lambda i: (0,) * len(shape))
    out_spec = pl.BlockSpec((BB, L, ENC), lambda i: (i, 0, 0))
    out_sds = jax.ShapeDtypeStruct((B, L, ENC), jnp.float32)
    return pl.pallas_call(
        _body,
        grid=(GRID,),
        in_specs=[
            pl.BlockSpec((1, M), lambda i: (0, i)),
            full((N_FAM, SED)), full((N_FAM, SED)),
            full((ENC, SED)), full((1, ENC)),
            full((ENC, ENC)), full((1, ENC)),
            full((ENC, ENC)), full((1, ENC)),
            full((ENC, ENC)), full((1, ENC)),
        ],
        out_specs=[out_spec, out_spec, out_spec],
        out_shape=[out_sds, out_sds, out_sds],
        compiler_params=pltpu.CompilerParams(
            dimension_semantics=("parallel",)),
    )(ids_pad, fe, rb, w1, b1, w2, b2, ws, bs, wsh, bsh)


def kernel(script_ids, family_embed, retroflex_bias, W1, b1, W2, b2, Ws, bs, Wsh, bsh):
    ids_pad = jnp.pad(script_ids.astype(jnp.int32), ((0, 0), (0, LP - L)),
                      constant_values=N_FAM).reshape(1, B * LP)
    proj, scale, shift = _run(
        ids_pad, family_embed, retroflex_bias, W1, b1.reshape(1, ENC),
        W2, b2.reshape(1, ENC), Ws, bs.reshape(1, ENC),
        Wsh, bsh.reshape(1, ENC))
    return (proj, scale, shift)


# speedup vs baseline: 22.7144x; 3.4800x over previous
import functools

import jax
import jax.numpy as jnp
from jax import lax
from jax.experimental import pallas as pl
from jax.experimental.pallas import tpu as pltpu

N_FAM = 12
SED = 32
ENC = 128
B, L = 16384, 20
MB = 8192
K = B // MB
GRID = L * K


def _body(ids_ref, fe_ref, rb_ref, w1_ref, b1_ref, w2_ref, b2_ref,
          ws_ref, bs_ref, wsh_ref, bsh_ref,
          proj_ref, scale_ref, shift_ref):
    raw = fe_ref[...] + rb_ref[...]
    h = lax.dot_general(raw, w1_ref[...], (((1,), (1,)), ((), ())),
                        preferred_element_type=jnp.float32) + b1_ref[...]
    h = h * jax.nn.sigmoid(h)
    p = lax.dot_general(h, w2_ref[...], (((1,), (1,)), ((), ())),
                        preferred_element_type=jnp.float32) + b2_ref[...]
    s = lax.dot_general(p, ws_ref[...], (((1,), (1,)), ((), ())),
                        preferred_element_type=jnp.float32) + bs_ref[...]
    sh = lax.dot_general(p, wsh_ref[...], (((1,), (1,)), ((), ())),
                         preferred_element_type=jnp.float32) + bsh_ref[...]
    table = jnp.concatenate([p, s, sh], axis=1)

    ids = ids_ref[...]
    iota = lax.broadcasted_iota(jnp.int32, (N_FAM, MB), 0)
    oh = jnp.where(ids == iota, 1.0, 0.0).astype(jnp.float32)
    res = lax.dot_general(oh, table, (((0,), (0,)), ((), ())),
                          preferred_element_type=jnp.float32)
    proj_ref[...] = res[:, 0:ENC].reshape(1, MB, ENC)
    scale_ref[...] = res[:, ENC:2 * ENC].reshape(1, MB, ENC)
    shift_ref[...] = res[:, 2 * ENC:3 * ENC].reshape(1, MB, ENC)


@jax.jit
def _run(ids_lm, fe, rb, w1, b1, w2, b2, ws, bs, wsh, bsh):
    full = lambda shape: pl.BlockSpec(shape, lambda i: (0,) * len(shape))
    out_spec = pl.BlockSpec((1, MB, ENC), lambda i: (i // K, i % K, 0))
    out_sds = jax.ShapeDtypeStruct((L, B, ENC), jnp.float32)
    return pl.pallas_call(
        _body,
        grid=(GRID,),
        in_specs=[
            pl.BlockSpec((1, MB), lambda i: (0, i)),
            full((N_FAM, SED)), full((N_FAM, SED)),
            full((ENC, SED)), full((1, ENC)),
            full((ENC, ENC)), full((1, ENC)),
            full((ENC, ENC)), full((1, ENC)),
            full((ENC, ENC)), full((1, ENC)),
        ],
        out_specs=[out_spec, out_spec, out_spec],
        out_shape=[out_sds, out_sds, out_sds],
        compiler_params=pltpu.CompilerParams(
            dimension_semantics=("parallel",)),
    )(ids_lm, fe, rb, w1, b1, w2, b2, ws, bs, wsh, bsh)


def kernel(script_ids, family_embed, retroflex_bias, W1, b1, W2, b2, Ws, bs, Wsh, bsh):
    ids_lm = script_ids.astype(jnp.int32).T.reshape(1, L * B)
    proj, scale, shift = _run(
        ids_lm, family_embed, retroflex_bias, W1, b1.reshape(1, ENC),
        W2, b2.reshape(1, ENC), Ws, bs.reshape(1, ENC),
        Wsh, bsh.reshape(1, ENC))
    tr = lambda x: jnp.transpose(x, (1, 0, 2))
    return (tr(proj), tr(scale), tr(shift))
